# baseline (speedup 1.0000x reference)
import functools

import jax
import jax.numpy as jnp
from jax.experimental import pallas as pl

TOPK = 8
CAP_FACTOR = 1.25
LB_W = 0.01
TB = 1024


def _route_body(x_ref, w_ref, probs_ref, disp_ref, psum_ref, tpe_ref,
                loss_ref):
    num_e = w_ref.shape[0]
    x = x_ref[...]
    w = w_ref[...]
    logits = jax.lax.dot_general(
        x, w, (((1,), (1,)), ((), ())), preferred_element_type=jnp.float32)
    mx = jnp.max(logits, axis=-1, keepdims=True)
    ex = jnp.exp(logits - mx)
    p = ex / jnp.sum(ex, axis=-1, keepdims=True)
    probs_ref[...] = p

    lane = jax.lax.broadcasted_iota(jnp.int32, p.shape, 1)
    work = p
    ssum = jnp.zeros((p.shape[0], 1), jnp.float32)
    for _ in range(TOPK):
        m = jnp.max(work, axis=-1, keepdims=True)
        cand = jnp.where(work == m, lane, num_e)
        sel = jnp.min(cand, axis=-1, keepdims=True)
        work = jnp.where(lane == sel, -1.0, work)
        ssum = ssum + m
    d = jnp.where(work < 0.0, p, 0.0) / ssum
    disp_ref[...] = d

    part_p = jnp.sum(p, axis=0, keepdims=True)
    part_t = jnp.sum(d, axis=0, keepdims=True)
    i = pl.program_id(0)

    @pl.when(i == 0)
    def _():
        psum_ref[...] = part_p
        tpe_ref[...] = part_t

    @pl.when(i != 0)
    def _():
        psum_ref[...] += part_p
        tpe_ref[...] += part_t

    @pl.when(i == pl.num_programs(0) - 1)
    def _():
        n_tok = pl.num_programs(0) * x_ref.shape[0]
        loss = jnp.sum(psum_ref[...] * tpe_ref[...]) * jnp.float32(
            LB_W / n_tok)
        loss_ref[...] = loss.reshape(1, 1)


def _cap_body(disp_ref, tpe_ref, out_ref, *, capacity):
    n_tok, num_e = disp_ref.shape
    tpe = tpe_ref[...]
    apply_drop = tpe > jnp.float32(capacity)

    m_val = disp_ref[...]
    m_bits = jax.lax.bitcast_convert_type(m_val, jnp.int32)

    def cnt_ge(t):
        return jnp.sum((m_bits >= t).astype(jnp.int32), axis=0, keepdims=True)

    one_bits = jax.lax.bitcast_convert_type(
        jnp.full((1, num_e), 1.0, jnp.float32), jnp.int32)
    lo0 = jnp.zeros((1, num_e), jnp.int32)
    hi0 = one_bits + 1

    def bs_body(_, lh):
        lo, hi = lh
        mid = (lo + hi) >> 1
        ge = cnt_ge(mid) >= capacity
        return jnp.where(ge, mid, lo), jnp.where(ge, hi, mid)

    vstar, _ = jax.lax.fori_loop(0, 31, bs_body, (lo0, hi0))
    c_gt = cnt_ge(vstar + 1)
    n_eq = capacity - c_gt

    row = jax.lax.broadcasted_iota(jnp.int32, (n_tok, num_e), 0)
    eq = m_bits == vstar

    def cnt_eq_le(i):
        return jnp.sum((eq & (row <= i)).astype(jnp.int32), axis=0,
                       keepdims=True)

    lo_i0 = jnp.full((1, num_e), -1, jnp.int32)
    hi_i0 = jnp.full((1, num_e), n_tok - 1, jnp.int32)

    def bsi_body(_, lh):
        lo, hi = lh
        mid = (lo + hi) >> 1
        ok = cnt_eq_le(mid) >= n_eq
        return jnp.where(ok, lo, mid), jnp.where(ok, mid, hi)

    _, istar = jax.lax.fori_loop(0, 15, bsi_body, (lo_i0, hi_i0))

    keep = (m_bits > vstar) | (eq & (row <= istar))
    out_ref[...] = jnp.where(keep | ~apply_drop, m_val, 0.0)


def _route_call(x, router_w, n_tok, d, num_e):
    return pl.pallas_call(
        _route_body,
        grid=(n_tok // TB,),
        in_specs=[
            pl.BlockSpec((TB, d), lambda i: (i, 0)),
            pl.BlockSpec((num_e, d), lambda i: (0, 0)),
        ],
        out_specs=[
            pl.BlockSpec((TB, num_e), lambda i: (i, 0)),
            pl.BlockSpec((TB, num_e), lambda i: (i, 0)),
            pl.BlockSpec((1, num_e), lambda i: (0, 0)),
            pl.BlockSpec((1, num_e), lambda i: (0, 0)),
            pl.BlockSpec((1, 1), lambda i: (0, 0)),
        ],
        out_shape=[
            jax.ShapeDtypeStruct((n_tok, num_e), jnp.float32),
            jax.ShapeDtypeStruct((n_tok, num_e), jnp.float32),
            jax.ShapeDtypeStruct((1, num_e), jnp.float32),
            jax.ShapeDtypeStruct((1, num_e), jnp.float32),
            jax.ShapeDtypeStruct((1, 1), jnp.float32),
        ],
    )(x, router_w)


def _cap_call(disp, tpe, n_tok, num_e, capacity):
    return pl.pallas_call(
        functools.partial(_cap_body, capacity=capacity),
        in_specs=[
            pl.BlockSpec((n_tok, num_e), lambda: (0, 0)),
            pl.BlockSpec((1, num_e), lambda: (0, 0)),
        ],
        out_specs=pl.BlockSpec((n_tok, num_e), lambda: (0, 0)),
        out_shape=jax.ShapeDtypeStruct((n_tok, num_e), jnp.float32),
    )(disp, tpe)


def kernel(hidden_states, router_w):
    b, s, d = hidden_states.shape
    num_e = router_w.shape[0]
    n_tok = b * s
    capacity = int(CAP_FACTOR * s * b / num_e)
    x = hidden_states.reshape(n_tok, d)

    probs, disp, _psum, tpe, loss = _route_call(x, router_w, n_tok, d, num_e)

    any_over = jnp.any(tpe > jnp.float32(capacity))
    dropped = jax.lax.cond(
        any_over,
        lambda dm, t: _cap_call(dm, t, n_tok, num_e, capacity),
        lambda dm, t: dm,
        disp, tpe)

    d_out = dropped.reshape(b, s, num_e)
    return d_out, d_out, loss.reshape(()), probs.reshape(b, s, num_e)
